# baseline (speedup 1.0000x reference)
import jax
import jax.numpy as jnp
from jax import lax
from jax.experimental import pallas as pl
from jax.experimental.pallas import tpu as pltpu
from jax.experimental.pallas import tpu_sc as plsc

N = 50000
D = 128
E = 200000
NREL = 3

NC, NS, L = 2, 16, 16
NP = 12544
NRANGE = 4
RPS = NRANGE // NC
DW = 144
ET = 13312
CH = 832
NCHUNK = ET // CH
GB = 32
MB = CH + GB
RPT = NP // NS
DUMMY = NP
NPAD = NRANGE * NP


def _sc_body(xa, edges3, zrows, agg, ebuf0, esem0, ebuf1, esem1, fsrc, foff,
             sidx0, soff0, rows0, sem0, sidx1, soff1, rows1, sem1, acc):
    c = lax.axis_index("c")
    s = lax.axis_index("s")
    base = s * RPT

    def fire(b, sidxS, soffS, rowsS, semS):
        for j in range(GB // L):
            sidxS[pl.ds(j * L, L)] = fsrc[pl.ds(b * GB + j * L, L)]
            soffS[pl.ds(j * L, L)] = foff[pl.ds(b * GB + j * L, L)]
        pltpu.async_copy(xa.at[sidxS], rowsS, semS)

    def finish(sidxS, soffS, rowsS, semS):
        pltpu.make_async_copy(xa.at[sidxS], rowsS, semS).wait()
        pltpu.sync_copy(rowsS, acc.at[soffS], add=True)

    def drain(cnt):
      with jax.named_scope("drain"):
        nb = cnt // GB

        @pl.when(nb > 0)
        def _():
            fire(0, sidx0, soff0, rows0, sem0)

        def gpair(g, _):
            b1 = 2 * g + 1

            @pl.when(b1 < nb)
            def _():
                fire(b1, sidx1, soff1, rows1, sem1)

            finish(sidx0, soff0, rows0, sem0)

            @pl.when(b1 < nb)
            def _():
                @pl.when(b1 + 1 < nb)
                def _():
                    fire(b1 + 1, sidx0, soff0, rows0, sem0)

                finish(sidx1, soff1, rows1, sem1)

            return 0

        lax.fori_loop(0, (nb + 1) // 2, gpair, 0)
        for j in range(GB // L):
            fsrc[pl.ds(j * L, L)] = fsrc[pl.ds(nb * GB + j * L, L)]
            foff[pl.ds(j * L, L)] = foff[pl.ds(nb * GB + j * L, L)]
        return cnt - nb * GB

    def rng_body(rel, rng):
        rg = RPS * c + rng
        lo = rg * NP

        with jax.named_scope("zero"):
            pltpu.sync_copy(zrows, acc.at[pl.ds(base, RPT)])
            plsc.subcore_barrier()

        def scan_chunk(ebufS, cnt):
          with jax.named_scope("edge_scan"):
            for k in range(CH // L):
                dv = ebufS[0, pl.ds(k * L, L)]
                sv = ebufS[1, pl.ds(k * L, L)]
                m = (dv >= lo) & (dv < lo + NP)
                mi = m.astype(jnp.int32)
                pos = cnt + jnp.cumsum(mi) - 1
                plsc.store_scatter(foff, [pos], dv - lo, mask=m)
                plsc.store_scatter(fsrc, [pos], sv, mask=m)
                cnt = cnt + plsc.all_reduce_population_count(m)[0]
            return cnt

        def load_e(ch, ebufS, esemS):
            pltpu.async_copy(edges3.at[rel, s, ch], ebufS, esemS)

        def wait_e(ebufS, esemS):
            pltpu.make_async_copy(edges3.at[rel, s, 0], ebufS, esemS).wait()

        load_e(0, ebuf0, esem0)
        load_e(1, ebuf1, esem1)

        def pair_body(g, cnt):
            wait_e(ebuf0, esem0)
            cnt = scan_chunk(ebuf0, cnt)

            @pl.when(2 * g + 2 < NCHUNK)
            def _():
                load_e(2 * g + 2, ebuf0, esem0)

            cnt = drain(cnt)
            wait_e(ebuf1, esem1)
            cnt = scan_chunk(ebuf1, cnt)

            @pl.when(2 * g + 3 < NCHUNK)
            def _():
                load_e(2 * g + 3, ebuf1, esem1)

            return drain(cnt)

        cnt = lax.fori_loop(0, NCHUNK // 2, pair_body, 0)

        dummyv = jnp.full((L,), DUMMY, jnp.int32)
        zerov = jnp.zeros((L,), jnp.int32)
        for j in range(GB // L):
            foff[pl.ds(cnt + j * L, L)] = dummyv
            fsrc[pl.ds(cnt + j * L, L)] = zerov
        fire(0, sidx0, soff0, rows0, sem0)
        finish(sidx0, soff0, rows0, sem0)
        plsc.subcore_barrier()

        with jax.named_scope("wout"):
            pltpu.sync_copy(acc.at[pl.ds(base, RPT)],
                            agg.at[rel, rg, pl.ds(base, RPT)])
            plsc.subcore_barrier()
        return 0

    def rel_body(rel, _):
        lax.fori_loop(0, RPS, lambda rng, __: rng_body(rel, rng), 0)
        return 0

    lax.fori_loop(0, NREL, rel_body, 0)


def _sc_aggregate(xa, edges3, zrows):
    mesh = plsc.VectorSubcoreMesh(core_axis_name="c", subcore_axis_name="s")
    return pl.kernel(
        _sc_body,
        out_type=jax.ShapeDtypeStruct((NREL, NRANGE, NP, DW), jnp.float32),
        mesh=mesh,
        compiler_params=pltpu.CompilerParams(
            use_tc_tiling_on_sc=False, needs_layout_passes=False),
        scratch_types=[
            pltpu.VMEM((2, CH), jnp.int32),
            pltpu.SemaphoreType.DMA,
            pltpu.VMEM((2, CH), jnp.int32),
            pltpu.SemaphoreType.DMA,
            pltpu.VMEM((MB,), jnp.int32),
            pltpu.VMEM((MB,), jnp.int32),
            pltpu.VMEM((GB,), jnp.int32),
            pltpu.VMEM((GB,), jnp.int32),
            pltpu.VMEM((GB, DW), jnp.float32),
            pltpu.SemaphoreType.DMA,
            pltpu.VMEM((GB,), jnp.int32),
            pltpu.VMEM((GB,), jnp.int32),
            pltpu.VMEM((GB, DW), jnp.float32),
            pltpu.SemaphoreType.DMA,
            pltpu.VMEM_SHARED((NP + L, DW), jnp.float32),
        ],
    )(xa, edges3, zrows)


RB = 448
TGRID = NPAD // RB
BPR = NP // RB


def _tc_body(xa_ref, agg_hbm, wt_ref, b_ref, out_ref, ab0, ab1, as0, as1):
    i = pl.program_id(0)

    def fire(j, abS, asS):
        for r in range(NREL):
            pltpu.make_async_copy(
                agg_hbm.at[r, j // BPR, pl.ds((j % BPR) * RB, RB)],
                abS.at[r], asS).start()

    def wait(abS, asS):
        for r in range(NREL):
            pltpu.make_async_copy(
                agg_hbm.at[r, 0, pl.ds(0, RB)], abS.at[r], asS).wait()

    def compute(abS):
        xb = xa_ref[:, :D]
        acc = jnp.dot(xb, wt_ref[0], preferred_element_type=jnp.float32)
        acc += b_ref[0, :][None, :]
        for r in range(NREL):
            ar = abS[r]
            cntc = ar[:, D:D + 1]
            scale = 1.0 / jnp.maximum(cntc, 1.0)
            acc += jnp.dot(ar[:, :D] * scale, wt_ref[r + 1],
                           preferred_element_type=jnp.float32)
            acc += jnp.where(cntc > 0.0, 1.0, 0.0) * b_ref[r + 1, :][None, :]
        out_ref[...] = acc

    @pl.when(i == 0)
    def _():
        fire(0, ab0, as0)

    @pl.when(i % 2 == 0)
    def _():
        @pl.when(i + 1 < TGRID)
        def _():
            fire(i + 1, ab1, as1)

        wait(ab0, as0)
        compute(ab0)

    @pl.when(i % 2 == 1)
    def _():
        @pl.when(i + 1 < TGRID)
        def _():
            fire(i + 1, ab0, as0)

        wait(ab1, as1)
        compute(ab1)


def _tc_combine(xa, agg, wt_all, b_all):
    return pl.pallas_call(
        _tc_body,
        grid=(TGRID,),
        in_specs=[
            pl.BlockSpec((RB, DW), lambda i: (i, 0)),
            pl.BlockSpec(memory_space=pl.ANY),
            pl.BlockSpec((NREL + 1, D, D), lambda i: (0, 0, 0)),
            pl.BlockSpec((NREL + 1, D), lambda i: (0, 0)),
        ],
        out_specs=pl.BlockSpec((RB, D), lambda i: (i, 0)),
        out_shape=jax.ShapeDtypeStruct((NPAD, D), jnp.float32),
        scratch_shapes=[
            pltpu.VMEM((NREL, RB, DW), jnp.float32),
            pltpu.VMEM((NREL, RB, DW), jnp.float32),
            pltpu.SemaphoreType.DMA,
            pltpu.SemaphoreType.DMA,
        ],
    )(xa, agg, wt_all, b_all)


def kernel(x, edge_index_r0, edge_index_r1, edge_index_r2, W_r0, b_r0,
           W_r1, b_r1, W_r2, b_r2, W_self, b_self):
    xa = jnp.zeros((NPAD, DW), jnp.float32)
    xa = xa.at[:N, :D].set(x)
    xa = xa.at[:, D].set(1.0)

    pad = NS * ET - E
    rels = []
    for e in (edge_index_r0, edge_index_r1, edge_index_r2):
        sp = jnp.pad(e[0], (0, pad)).reshape(NS, NCHUNK, CH)
        dp = jnp.pad(e[1], (0, pad), constant_values=-1).reshape(NS, NCHUNK, CH)
        rels.append(jnp.stack([dp, sp], axis=2))
    edges3 = jnp.stack(rels)
    zrows = jnp.zeros((RPT, DW), jnp.float32)

    agg = _sc_aggregate(xa, edges3, zrows)

    wt_all = jnp.stack([W_self.T, W_r0.T, W_r1.T, W_r2.T])
    b_all = jnp.stack([b_self, b_r0, b_r1, b_r2])
    out = _tc_combine(xa, agg, wt_all, b_all)
    return out[:N]


# speedup vs baseline: 1.4421x; 1.4421x over previous
import jax
import jax.numpy as jnp
from jax import lax
from jax.experimental import pallas as pl
from jax.experimental.pallas import tpu as pltpu
from jax.experimental.pallas import tpu_sc as plsc

N = 50000
D = 128
E = 200000
NREL = 3

NC, NS, L = 2, 16, 16
NP = 12544
NRANGE = 4
RPS = NRANGE // NC
CW = 16
ET = 13312
CH = 1024
NCHUNK = ET // CH
GB = 32
MB = CH + GB
RPT = NP // NS
DUMMY = NP
NPAD = NRANGE * NP


def _sc_body(x, edges3, zrows, zcnt, aggf, cntf, ebuf0, esem0, ebuf1, esem1,
             fsrc, foff, sidx0, soff0, rows0, sem0, sidx1, soff1, rows1, sem1,
             ones, accf, accc):
    c = lax.axis_index("c")
    s = lax.axis_index("s")
    base = s * RPT

    one16 = jnp.ones((L,), jnp.float32)
    for r in range(GB):
        ones[r, pl.ds(0, CW)] = one16

    def fire(b, sidxS, soffS, rowsS, semS):
        for j in range(GB // L):
            sidxS[pl.ds(j * L, L)] = fsrc[pl.ds(b * GB + j * L, L)]
            soffS[pl.ds(j * L, L)] = foff[pl.ds(b * GB + j * L, L)]
        pltpu.async_copy(x.at[sidxS], rowsS, semS)
        pltpu.sync_copy(ones, accc.at[soffS], add=True)

    def finish(sidxS, soffS, rowsS, semS):
        pltpu.make_async_copy(x.at[sidxS], rowsS, semS).wait()
        pltpu.sync_copy(rowsS, accf.at[soffS], add=True)

    def drain(cnt):
      with jax.named_scope("drain"):
        nb = cnt // GB

        @pl.when(nb > 0)
        def _():
            fire(0, sidx0, soff0, rows0, sem0)

        def gpair(g, _):
            b1 = 2 * g + 1

            @pl.when(b1 < nb)
            def _():
                fire(b1, sidx1, soff1, rows1, sem1)

            finish(sidx0, soff0, rows0, sem0)

            @pl.when(b1 < nb)
            def _():
                @pl.when(b1 + 1 < nb)
                def _():
                    fire(b1 + 1, sidx0, soff0, rows0, sem0)

                finish(sidx1, soff1, rows1, sem1)

            return 0

        lax.fori_loop(0, (nb + 1) // 2, gpair, 0)
        for j in range(GB // L):
            fsrc[pl.ds(j * L, L)] = fsrc[pl.ds(nb * GB + j * L, L)]
            foff[pl.ds(j * L, L)] = foff[pl.ds(nb * GB + j * L, L)]
        return cnt - nb * GB

    def rng_body(rel, rng):
        rg = RPS * c + rng
        lo = rg * NP

        with jax.named_scope("zero"):
            pltpu.sync_copy(zrows, accf.at[pl.ds(base, RPT)])
            pltpu.sync_copy(zcnt, accc.at[pl.ds(base, RPT)])
            plsc.subcore_barrier()

        def scan_chunk(ebufS, cnt):
          with jax.named_scope("edge_scan"):
            for k in range(CH // L):
                dv = ebufS[0, k // 8, pl.ds((k % 8) * L, L)]
                sv = ebufS[1, k // 8, pl.ds((k % 8) * L, L)]
                m = (dv >= lo) & (dv < lo + NP)
                mi = m.astype(jnp.int32)
                pos = cnt + jnp.cumsum(mi) - 1
                plsc.store_scatter(foff, [pos], dv - lo, mask=m)
                plsc.store_scatter(fsrc, [pos], sv, mask=m)
                cnt = cnt + plsc.all_reduce_population_count(m)[0]
            return cnt

        def load_e(ch, ebufS, esemS):
            pltpu.async_copy(edges3.at[rel, s, ch], ebufS, esemS)

        def wait_e(ebufS, esemS):
            pltpu.make_async_copy(edges3.at[rel, s, 0], ebufS, esemS).wait()

        load_e(0, ebuf0, esem0)
        load_e(1, ebuf1, esem1)

        def pair_body(g, cnt):
            wait_e(ebuf0, esem0)
            cnt = scan_chunk(ebuf0, cnt)

            @pl.when(2 * g + 2 < NCHUNK)
            def _():
                load_e(2 * g + 2, ebuf0, esem0)

            cnt = drain(cnt)
            wait_e(ebuf1, esem1)
            cnt = scan_chunk(ebuf1, cnt)

            @pl.when(2 * g + 3 < NCHUNK)
            def _():
                load_e(2 * g + 3, ebuf1, esem1)

            return drain(cnt)

        cnt = lax.fori_loop(0, NCHUNK // 2, pair_body, 0)
        wait_e(ebuf0, esem0)
        cnt = scan_chunk(ebuf0, cnt)
        cnt = drain(cnt)

        dummyv = jnp.full((L,), DUMMY, jnp.int32)
        zerov = jnp.zeros((L,), jnp.int32)
        for j in range(GB // L):
            foff[pl.ds(cnt + j * L, L)] = dummyv
            fsrc[pl.ds(cnt + j * L, L)] = zerov
        fire(0, sidx0, soff0, rows0, sem0)
        finish(sidx0, soff0, rows0, sem0)
        plsc.subcore_barrier()

        with jax.named_scope("wout"):
            pltpu.sync_copy(accf.at[pl.ds(base, RPT)],
                            aggf.at[rel, rg, pl.ds(base, RPT)])
            pltpu.sync_copy(accc.at[pl.ds(base, RPT)],
                            cntf.at[rel, rg, pl.ds(base, RPT)])
            plsc.subcore_barrier()
        return 0

    def rel_body(rel, _):
        lax.fori_loop(0, RPS, lambda rng, __: rng_body(rel, rng), 0)
        return 0

    lax.fori_loop(0, NREL, rel_body, 0)


def _sc_aggregate(x, edges3, zrows, zcnt):
    mesh = plsc.VectorSubcoreMesh(core_axis_name="c", subcore_axis_name="s")
    return pl.kernel(
        _sc_body,
        out_type=(
            jax.ShapeDtypeStruct((NREL, NRANGE, NP, D), jnp.float32),
            jax.ShapeDtypeStruct((NREL, NRANGE, NP, CW), jnp.float32),
        ),
        mesh=mesh,
        compiler_params=pltpu.CompilerParams(
            use_tc_tiling_on_sc=False, needs_layout_passes=False),
        scratch_types=[
            pltpu.VMEM((2, CH // 128, 128), jnp.int32),
            pltpu.SemaphoreType.DMA,
            pltpu.VMEM((2, CH // 128, 128), jnp.int32),
            pltpu.SemaphoreType.DMA,
            pltpu.VMEM((MB,), jnp.int32),
            pltpu.VMEM((MB,), jnp.int32),
            pltpu.VMEM((GB,), jnp.int32),
            pltpu.VMEM((GB,), jnp.int32),
            pltpu.VMEM((GB, D), jnp.float32),
            pltpu.SemaphoreType.DMA,
            pltpu.VMEM((GB,), jnp.int32),
            pltpu.VMEM((GB,), jnp.int32),
            pltpu.VMEM((GB, D), jnp.float32),
            pltpu.SemaphoreType.DMA,
            pltpu.VMEM((GB, CW), jnp.float32),
            pltpu.VMEM_SHARED((NP + L, D), jnp.float32),
            pltpu.VMEM_SHARED((NP + L, CW), jnp.float32),
        ],
    )(x, edges3, zrows, zcnt)


RB = 400
TGRID = N // RB
CB = RB // 8


def _tc_body(x_ref, agg_ref, cnt_ref, wt_ref, b_ref, out_ref):
    i = pl.program_id(0)
    xb = x_ref[...]
    acc = jnp.dot(xb, wt_ref[0], preferred_element_type=jnp.float32)
    acc += b_ref[0, :][None, :]
    rowmod = lax.broadcasted_iota(jnp.int32, (RB, D), 0) % 8
    colidx = lax.broadcasted_iota(jnp.int32, (RB, D), 1)
    pick = colidx == rowmod * CW
    for r in range(NREL):
        ar = agg_ref[r]
        P = cnt_ref[r, pl.ds(i * CB, CB)]
        Pr = jnp.broadcast_to(P[:, None, :], (CB, 8, D)).reshape(RB, D)
        cntc = jnp.sum(jnp.where(pick, Pr, 0.0), axis=1, keepdims=True)
        scale = 1.0 / jnp.maximum(cntc, 1.0)
        acc += jnp.dot(ar * scale, wt_ref[r + 1],
                       preferred_element_type=jnp.float32)
        acc += jnp.where(cntc > 0.0, 1.0, 0.0) * b_ref[r + 1, :][None, :]
    out_ref[...] = acc


def _tc_combine(x, aggv, cntv, wt_all, b_all):
    return pl.pallas_call(
        _tc_body,
        grid=(TGRID,),
        in_specs=[
            pl.BlockSpec((RB, D), lambda i: (i, 0)),
            pl.BlockSpec((NREL, RB, D), lambda i: (0, i, 0)),
            pl.BlockSpec((NREL, NPAD * CW // 128, D), lambda i: (0, 0, 0)),
            pl.BlockSpec((NREL + 1, D, D), lambda i: (0, 0, 0)),
            pl.BlockSpec((NREL + 1, D), lambda i: (0, 0)),
        ],
        out_specs=pl.BlockSpec((RB, D), lambda i: (i, 0)),
        out_shape=jax.ShapeDtypeStruct((N, D), jnp.float32),
    )(x, aggv, cntv, wt_all, b_all)


def kernel(x, edge_index_r0, edge_index_r1, edge_index_r2, W_r0, b_r0,
           W_r1, b_r1, W_r2, b_r2, W_self, b_self):
    pad = NS * ET - E
    rels = []
    for e in (edge_index_r0, edge_index_r1, edge_index_r2):
        sp = jnp.pad(e[0], (0, pad)).reshape(NS, NCHUNK, CH // 128, 128)
        dp = jnp.pad(e[1], (0, pad),
                     constant_values=-1).reshape(NS, NCHUNK, CH // 128, 128)
        rels.append(jnp.stack([dp, sp], axis=2))
    edges3 = jnp.stack(rels)
    zrows = jnp.zeros((RPT, D), jnp.float32)
    zcnt = jnp.zeros((RPT, CW), jnp.float32)

    aggf, cntf = _sc_aggregate(x, edges3, zrows, zcnt)
    aggv = aggf.reshape(NREL, NPAD, D)
    cntv = cntf.reshape(NREL, NPAD * CW // 128, 128)

    wt_all = jnp.stack([W_self.T, W_r0.T, W_r1.T, W_r2.T])
    b_all = jnp.stack([b_self, b_r0, b_r1, b_r2])
    return _tc_combine(x, aggv, cntv, wt_all, b_all)
